# Initial kernel scaffold; baseline (speedup 1.0000x reference)
import functools

import jax
import jax.numpy as jnp
from jax import lax
from jax.experimental import pallas as pl
from jax.experimental.pallas import tpu as pltpu
from jax.experimental.pallas import tpu_sc as plsc

N, E, D = 10000, 320000, 128
NC, NS = 2, 16
NW = NC * NS
EPW = E // NW
C = 80
NCHUNK = EPW // C
TPR = N // NS
ZR = 125

_mesh = plsc.VectorSubcoreMesh(core_axis_name="c", subcore_axis_name="s")


def _sc_agg(h, src, dst):
    @functools.partial(
        pl.kernel,
        out_type=jax.ShapeDtypeStruct((NC, N, D), jnp.float32),
        mesh=_mesh,
        scratch_types=[
            pltpu.VMEM((C,), jnp.int32),
            pltpu.VMEM((C,), jnp.int32),
            pltpu.VMEM((C, D), jnp.float32),
            pltpu.VMEM((ZR, D), jnp.float32),
            pltpu.VMEM_SHARED((N, D), jnp.float32),
            pltpu.SemaphoreType.DMA,
        ],
    )
    def k(h_hbm, src_hbm, dst_hbm, out_hbm, sidx, didx, rows, zbuf, agg_sh, sem):
        cid = lax.axis_index("c")
        sid = lax.axis_index("s")
        wid = sid * NC + cid

        z16 = jnp.zeros((16,), jnp.float32)

        @pl.loop(0, ZR)
        def _zero(r):
            for j in range(D // 16):
                zbuf[r, pl.ds(j * 16, 16)] = z16

        row0 = sid * TPR
        for t in range(TPR // ZR):
            pltpu.sync_copy(zbuf, agg_sh.at[pl.ds(row0 + t * ZR, ZR)])
        plsc.subcore_barrier()

        base_w = wid * EPW

        @pl.loop(0, NCHUNK)
        def _edges(i):
            base = base_w + i * C
            pltpu.sync_copy(src_hbm.at[pl.ds(base, C)], sidx)
            pltpu.sync_copy(dst_hbm.at[pl.ds(base, C)], didx)
            pltpu.async_copy(h_hbm.at[sidx], rows, sem).wait()
            pltpu.sync_copy(rows, agg_sh.at[didx], add=True)

        plsc.subcore_barrier()
        pltpu.sync_copy(agg_sh.at[pl.ds(row0, TPR)],
                        out_hbm.at[cid, pl.ds(row0, TPR)])

    return k(h, src, dst)


BN = 1000


def _leaky(v):
    return jnp.where(v > 0, v, 0.2 * v)


def _mlp_body(h_ref, agg_ref, wa_ref, ba_ref, wb_ref, bb_ref, o_ref):
    z = h_ref[...] + agg_ref[0] + agg_ref[1]
    z = _leaky(jnp.dot(z, wa_ref[...], preferred_element_type=jnp.float32,
                       precision=lax.Precision.HIGHEST) + ba_ref[...])
    z = _leaky(jnp.dot(z, wb_ref[...], preferred_element_type=jnp.float32,
                       precision=lax.Precision.HIGHEST) + bb_ref[...])
    o_ref[...] = z


def _tc_mlp(h, agg, waT, ba, wbT, bb):
    return pl.pallas_call(
        _mlp_body,
        grid=(N // BN,),
        in_specs=[
            pl.BlockSpec((BN, D), lambda i: (i, 0)),
            pl.BlockSpec((NC, BN, D), lambda i: (0, i, 0)),
            pl.BlockSpec((D, D), lambda i: (0, 0)),
            pl.BlockSpec((1, D), lambda i: (0, 0)),
            pl.BlockSpec((D, D), lambda i: (0, 0)),
            pl.BlockSpec((1, D), lambda i: (0, 0)),
        ],
        out_specs=pl.BlockSpec((BN, D), lambda i: (i, 0)),
        out_shape=jax.ShapeDtypeStruct((N, D), jnp.float32),
    )(h, agg, waT, ba, wbT, bb)


def _final_body(h_ref, agg_ref, wa_ref, ba_ref, wb_ref, bb_ref, w3_ref, b3_ref,
                o_ref):
    z = h_ref[...] + agg_ref[0] + agg_ref[1]
    z = _leaky(jnp.dot(z, wa_ref[...], preferred_element_type=jnp.float32,
                       precision=lax.Precision.HIGHEST) + ba_ref[...])
    z = _leaky(jnp.dot(z, wb_ref[...], preferred_element_type=jnp.float32,
                       precision=lax.Precision.HIGHEST) + bb_ref[...])
    o_ref[...] = jnp.sum(z * w3_ref[...], axis=1, keepdims=True) + b3_ref[...]


def _tc_final(h, agg, waT, ba, wbT, bb, w3, b3):
    return pl.pallas_call(
        _final_body,
        grid=(N // BN,),
        in_specs=[
            pl.BlockSpec((BN, D), lambda i: (i, 0)),
            pl.BlockSpec((NC, BN, D), lambda i: (0, i, 0)),
            pl.BlockSpec((D, D), lambda i: (0, 0)),
            pl.BlockSpec((1, D), lambda i: (0, 0)),
            pl.BlockSpec((D, D), lambda i: (0, 0)),
            pl.BlockSpec((1, D), lambda i: (0, 0)),
            pl.BlockSpec((1, D), lambda i: (0, 0)),
            pl.BlockSpec((1, 1), lambda i: (0, 0)),
        ],
        out_specs=pl.BlockSpec((BN, 1), lambda i: (i, 0)),
        out_shape=jax.ShapeDtypeStruct((N, 1), jnp.float32),
    )(h, agg, waT, ba, wbT, bb, w3, b3)


def kernel(x, adj, edge_index, W1a, b1a, W1b, b1b, W2a, b2a, W2b, b2b, W3, b3):
    src = edge_index[0]
    dst = edge_index[1]
    agg1 = _sc_agg(x, src, dst)
    h1 = _tc_mlp(x, agg1, W1a.T, b1a.reshape(1, D), W1b.T, b1b.reshape(1, D))
    agg2 = _sc_agg(h1, src, dst)
    return _tc_final(h1, agg2, W2a.T, b2a.reshape(1, D), W2b.T, b2b.reshape(1, D),
                     W3, b3.reshape(1, 1))


# speedup vs baseline: 4.8470x; 4.8470x over previous
import functools

import jax
import jax.numpy as jnp
from jax import lax
from jax.experimental import pallas as pl
from jax.experimental.pallas import tpu as pltpu
from jax.experimental.pallas import tpu_sc as plsc

N, E, D = 10000, 320000, 128
NC, NS = 2, 16
NW = NC * NS
EPW = E // NW
C = 80
NCHUNK = EPW // C
NP = 10240
TPR = NP // NS
ZR = 128

_mesh = plsc.VectorSubcoreMesh(core_axis_name="c", subcore_axis_name="s")


def _sc_agg(h, src, dst):
    @functools.partial(
        pl.kernel,
        out_type=jax.ShapeDtypeStruct((NC, NP, D), jnp.float32),
        mesh=_mesh,
        scratch_types=[
            pltpu.VMEM((C,), jnp.int32),
            pltpu.VMEM((C,), jnp.int32),
            pltpu.VMEM((C, D), jnp.float32),
            pltpu.VMEM((ZR, D), jnp.float32),
            pltpu.VMEM_SHARED((NP, D), jnp.float32),
            pltpu.SemaphoreType.DMA,
        ],
    )
    def k(h_hbm, src_hbm, dst_hbm, out_hbm, sidx, didx, rows, zbuf, agg_sh, sem):
        cid = lax.axis_index("c")
        sid = lax.axis_index("s")
        wid = sid * NC + cid

        z16 = jnp.zeros((16,), jnp.float32)

        @pl.loop(0, ZR)
        def _zero(r):
            for j in range(D // 16):
                zbuf[r, pl.ds(j * 16, 16)] = z16

        row0 = sid * TPR
        for t in range(TPR // ZR):
            pltpu.sync_copy(zbuf, agg_sh.at[pl.ds(row0 + t * ZR, ZR)])
        plsc.subcore_barrier()

        base_w = wid * EPW

        @pl.loop(0, NCHUNK)
        def _edges(i):
            base = base_w + i * C
            pltpu.sync_copy(src_hbm.at[pl.ds(base, C)], sidx)
            pltpu.sync_copy(dst_hbm.at[pl.ds(base, C)], didx)
            pltpu.async_copy(h_hbm.at[sidx], rows, sem).wait()
            pltpu.sync_copy(rows, agg_sh.at[didx], add=True)

        plsc.subcore_barrier()
        pltpu.sync_copy(agg_sh.at[pl.ds(row0, TPR)],
                        out_hbm.at[cid, pl.ds(row0, TPR)])

    return k(h, src, dst)


BN = 1000


def _leaky(v):
    return jnp.where(v > 0, v, 0.2 * v)


def _mlp_body(h_ref, agg_ref, wa_ref, ba_ref, wb_ref, bb_ref, o_ref):
    z = h_ref[...] + agg_ref[0] + agg_ref[1]
    z = _leaky(jnp.dot(z, wa_ref[...], preferred_element_type=jnp.float32,
                       precision=lax.Precision.HIGHEST) + ba_ref[...])
    z = _leaky(jnp.dot(z, wb_ref[...], preferred_element_type=jnp.float32,
                       precision=lax.Precision.HIGHEST) + bb_ref[...])
    o_ref[...] = z


def _tc_mlp(h, agg, waT, ba, wbT, bb):
    return pl.pallas_call(
        _mlp_body,
        grid=(N // BN,),
        in_specs=[
            pl.BlockSpec((BN, D), lambda i: (i, 0)),
            pl.BlockSpec((NC, BN, D), lambda i: (0, i, 0)),
            pl.BlockSpec((D, D), lambda i: (0, 0)),
            pl.BlockSpec((1, D), lambda i: (0, 0)),
            pl.BlockSpec((D, D), lambda i: (0, 0)),
            pl.BlockSpec((1, D), lambda i: (0, 0)),
        ],
        out_specs=pl.BlockSpec((BN, D), lambda i: (i, 0)),
        out_shape=jax.ShapeDtypeStruct((N, D), jnp.float32),
    )(h, agg, waT, ba, wbT, bb)


def _final_body(h_ref, agg_ref, wa_ref, ba_ref, wb_ref, bb_ref, w3_ref, b3_ref,
                o_ref):
    z = h_ref[...] + agg_ref[0] + agg_ref[1]
    z = _leaky(jnp.dot(z, wa_ref[...], preferred_element_type=jnp.float32,
                       precision=lax.Precision.HIGHEST) + ba_ref[...])
    z = _leaky(jnp.dot(z, wb_ref[...], preferred_element_type=jnp.float32,
                       precision=lax.Precision.HIGHEST) + bb_ref[...])
    o_ref[...] = jnp.sum(z * w3_ref[...], axis=1, keepdims=True) + b3_ref[...]


def _tc_final(h, agg, waT, ba, wbT, bb, w3, b3):
    return pl.pallas_call(
        _final_body,
        grid=(N // BN,),
        in_specs=[
            pl.BlockSpec((BN, D), lambda i: (i, 0)),
            pl.BlockSpec((NC, BN, D), lambda i: (0, i, 0)),
            pl.BlockSpec((D, D), lambda i: (0, 0)),
            pl.BlockSpec((1, D), lambda i: (0, 0)),
            pl.BlockSpec((D, D), lambda i: (0, 0)),
            pl.BlockSpec((1, D), lambda i: (0, 0)),
            pl.BlockSpec((1, D), lambda i: (0, 0)),
            pl.BlockSpec((1, 1), lambda i: (0, 0)),
        ],
        out_specs=pl.BlockSpec((BN, 1), lambda i: (i, 0)),
        out_shape=jax.ShapeDtypeStruct((N, 1), jnp.float32),
    )(h, agg, waT, ba, wbT, bb, w3, b3)


def kernel(x, adj, edge_index, W1a, b1a, W1b, b1b, W2a, b2a, W2b, b2b, W3, b3):
    src = edge_index[0]
    dst = edge_index[1]
    agg1 = _sc_agg(x, src, dst)
    h1 = _tc_mlp(x, agg1, W1a.T, b1a.reshape(1, D), W1b.T, b1b.reshape(1, D))
    agg2 = _sc_agg(h1, src, dst)
    return _tc_final(h1, agg2, W2a.T, b2a.reshape(1, D), W2b.T, b2b.reshape(1, D),
                     W3, b3.reshape(1, 1))


# speedup vs baseline: 10.1611x; 2.0964x over previous
import functools

import jax
import jax.numpy as jnp
from jax import lax
from jax.experimental import pallas as pl
from jax.experimental.pallas import tpu as pltpu
from jax.experimental.pallas import tpu_sc as plsc

N, E, D = 10000, 320000, 128
NC, NS = 2, 16
NW = NC * NS
EPW = E // NW
C = 80
NCHUNK = EPW // C
NP = 10240
TPR = NP // NS
ZR = 32

_mesh = plsc.VectorSubcoreMesh(core_axis_name="c", subcore_axis_name="s")


def _sc_agg(h, src, dst):
    @functools.partial(
        pl.kernel,
        out_type=jax.ShapeDtypeStruct((NC, NP, D), jnp.float32),
        mesh=_mesh,
        scratch_types=[
            pltpu.VMEM((EPW,), jnp.int32),
            pltpu.VMEM((EPW,), jnp.int32),
            pltpu.VMEM((C,), jnp.int32),
            pltpu.VMEM((C,), jnp.int32),
            pltpu.VMEM((C,), jnp.int32),
            pltpu.VMEM((C,), jnp.int32),
            pltpu.VMEM((C, D), jnp.float32),
            pltpu.VMEM((C, D), jnp.float32),
            pltpu.VMEM((ZR, D), jnp.float32),
            pltpu.VMEM_SHARED((NP, D), jnp.float32),
            pltpu.SemaphoreType.DMA,
            pltpu.SemaphoreType.DMA,
        ],
    )
    def k(h_hbm, src_hbm, dst_hbm, out_hbm, sidx, didx, s0, s1, d0, d1, r0, r1,
          zbuf, agg_sh, g0, g1):
        cid = lax.axis_index("c")
        sid = lax.axis_index("s")
        wid = sid * NC + cid
        base_w = wid * EPW

        pltpu.async_copy(src_hbm.at[pl.ds(base_w, EPW)], sidx, g0)
        pltpu.async_copy(dst_hbm.at[pl.ds(base_w, EPW)], didx, g1)

        z16 = jnp.zeros((16,), jnp.float32)

        @pl.loop(0, ZR)
        def _zero(r):
            for j in range(D // 16):
                zbuf[r, pl.ds(j * 16, 16)] = z16

        row0 = sid * TPR
        for t in range(TPR // ZR):
            pltpu.sync_copy(zbuf, agg_sh.at[pl.ds(row0 + t * ZR, ZR)])
        pltpu.make_async_copy(src_hbm.at[pl.ds(base_w, EPW)], sidx, g0).wait()
        pltpu.make_async_copy(dst_hbm.at[pl.ds(base_w, EPW)], didx, g1).wait()
        plsc.subcore_barrier()

        def copy_idx(i, sbuf, dbuf):
            base = i * C
            for j in range(C // 16):
                sbuf[pl.ds(j * 16, 16)] = sidx[pl.ds(base + j * 16, 16)]
                dbuf[pl.ds(j * 16, 16)] = didx[pl.ds(base + j * 16, 16)]

        def start_gather(sbuf, rbuf, sem):
            return pltpu.async_copy(h_hbm.at[sbuf], rbuf, sem)

        def wait_gather(sbuf, rbuf, sem):
            pltpu.make_async_copy(h_hbm.at[sbuf], rbuf, sem).wait()

        def scatter(rbuf, dbuf):
            pltpu.sync_copy(rbuf, agg_sh.at[dbuf], add=True)

        copy_idx(0, s0, d0)
        start_gather(s0, r0, g0)

        NG = NCHUNK // 2

        @pl.loop(0, NG)
        def _edges(g):
            a = 2 * g
            b = a + 1
            copy_idx(b, s1, d1)
            wait_gather(s0, r0, g0)
            start_gather(s1, r1, g1)
            scatter(r0, d0)

            @pl.when(g < NG - 1)
            def _():
                copy_idx(a + 2, s0, d0)
                start_gather(s0, r0, g0)

            wait_gather(s1, r1, g1)
            scatter(r1, d1)

        if NCHUNK % 2:
            copy_idx(NCHUNK - 1, s0, d0)
            start_gather(s0, r0, g0)
            wait_gather(s0, r0, g0)
            scatter(r0, d0)

        plsc.subcore_barrier()
        pltpu.sync_copy(agg_sh.at[pl.ds(row0, TPR)],
                        out_hbm.at[cid, pl.ds(row0, TPR)])

    return k(h, src, dst)


BN = 1000


def _leaky(v):
    return jnp.where(v > 0, v, 0.2 * v)


def _mlp_body(h_ref, agg_ref, wa_ref, ba_ref, wb_ref, bb_ref, o_ref):
    z = h_ref[...] + agg_ref[0] + agg_ref[1]
    z = _leaky(jnp.dot(z, wa_ref[...], preferred_element_type=jnp.float32,
                       precision=lax.Precision.HIGHEST) + ba_ref[...])
    z = _leaky(jnp.dot(z, wb_ref[...], preferred_element_type=jnp.float32,
                       precision=lax.Precision.HIGHEST) + bb_ref[...])
    o_ref[...] = z


def _tc_mlp(h, agg, waT, ba, wbT, bb):
    return pl.pallas_call(
        _mlp_body,
        grid=(N // BN,),
        in_specs=[
            pl.BlockSpec((BN, D), lambda i: (i, 0)),
            pl.BlockSpec((NC, BN, D), lambda i: (0, i, 0)),
            pl.BlockSpec((D, D), lambda i: (0, 0)),
            pl.BlockSpec((1, D), lambda i: (0, 0)),
            pl.BlockSpec((D, D), lambda i: (0, 0)),
            pl.BlockSpec((1, D), lambda i: (0, 0)),
        ],
        out_specs=pl.BlockSpec((BN, D), lambda i: (i, 0)),
        out_shape=jax.ShapeDtypeStruct((N, D), jnp.float32),
    )(h, agg, waT, ba, wbT, bb)


def _final_body(h_ref, agg_ref, wa_ref, ba_ref, wb_ref, bb_ref, w3_ref, b3_ref,
                o_ref):
    z = h_ref[...] + agg_ref[0] + agg_ref[1]
    z = _leaky(jnp.dot(z, wa_ref[...], preferred_element_type=jnp.float32,
                       precision=lax.Precision.HIGHEST) + ba_ref[...])
    z = _leaky(jnp.dot(z, wb_ref[...], preferred_element_type=jnp.float32,
                       precision=lax.Precision.HIGHEST) + bb_ref[...])
    o_ref[...] = jnp.sum(z * w3_ref[...], axis=1, keepdims=True) + b3_ref[...]


def _tc_final(h, agg, waT, ba, wbT, bb, w3, b3):
    return pl.pallas_call(
        _final_body,
        grid=(N // BN,),
        in_specs=[
            pl.BlockSpec((BN, D), lambda i: (i, 0)),
            pl.BlockSpec((NC, BN, D), lambda i: (0, i, 0)),
            pl.BlockSpec((D, D), lambda i: (0, 0)),
            pl.BlockSpec((1, D), lambda i: (0, 0)),
            pl.BlockSpec((D, D), lambda i: (0, 0)),
            pl.BlockSpec((1, D), lambda i: (0, 0)),
            pl.BlockSpec((1, D), lambda i: (0, 0)),
            pl.BlockSpec((1, 1), lambda i: (0, 0)),
        ],
        out_specs=pl.BlockSpec((BN, 1), lambda i: (i, 0)),
        out_shape=jax.ShapeDtypeStruct((N, 1), jnp.float32),
    )(h, agg, waT, ba, wbT, bb, w3, b3)


def kernel(x, adj, edge_index, W1a, b1a, W1b, b1b, W2a, b2a, W2b, b2b, W3, b3):
    src = edge_index[0]
    dst = edge_index[1]
    agg1 = _sc_agg(x, src, dst)
    h1 = _tc_mlp(x, agg1, W1a.T, b1a.reshape(1, D), W1b.T, b1b.reshape(1, D))
    agg2 = _sc_agg(h1, src, dst)
    return _tc_final(h1, agg2, W2a.T, b2a.reshape(1, D), W2b.T, b2b.reshape(1, D),
                     W3, b3.reshape(1, 1))


# speedup vs baseline: 10.3597x; 1.0195x over previous
import functools

import jax
import jax.numpy as jnp
from jax import lax
from jax.experimental import pallas as pl
from jax.experimental.pallas import tpu as pltpu
from jax.experimental.pallas import tpu_sc as plsc

N, E, D = 10000, 320000, 128
NC, NS = 2, 16
NW = NC * NS
EPW = E // NW
C = 80
NCHUNK = EPW // C
NP = 10240
K = 4
KI = 2 * K
NROUND = NCHUNK // KI

_mesh = plsc.VectorSubcoreMesh(core_axis_name="c", subcore_axis_name="s")


def _sc_agg(h, src, dst, zrows):
    @functools.partial(
        pl.kernel,
        out_type=jax.ShapeDtypeStruct((NC, NP, D), jnp.float32),
        mesh=_mesh,
        scratch_types=(
            [pltpu.VMEM((C,), jnp.int32)] * KI
            + [pltpu.VMEM((C,), jnp.int32)] * KI
            + [pltpu.VMEM((C, D), jnp.float32)] * K
            + [pltpu.VMEM_SHARED((NP, D), jnp.float32)]
            + [pltpu.SemaphoreType.DMA] * (1 + KI + 2 * K)
        ),
    )
    def k(h_hbm, src_hbm, dst_hbm, z_hbm, out_hbm, *refs):
        S = refs[0:KI]
        Dd = refs[KI:2 * KI]
        R = refs[2 * KI:2 * KI + K]
        agg_sh = refs[2 * KI + K]
        gz = refs[2 * KI + K + 1]
        base_sem = 2 * KI + K + 2
        I = refs[base_sem:base_sem + KI]
        G = refs[base_sem + KI:base_sem + KI + K]
        T = refs[base_sem + KI + K:base_sem + KI + 2 * K]

        cid = lax.axis_index("c")
        sid = lax.axis_index("s")
        wid = sid * NC + cid
        base_w = wid * EPW

        @pl.when(sid == 0)
        def _():
            pltpu.async_copy(z_hbm, agg_sh, gz)
            pltpu.make_async_copy(z_hbm, agg_sh, gz).wait()

        plsc.subcore_barrier()

        def fire_idx(cb, i):
            pltpu.async_copy(src_hbm.at[pl.ds(base_w + cb, C)], S[i], I[i])
            pltpu.async_copy(dst_hbm.at[pl.ds(base_w + cb, C)], Dd[i], I[i])

        def wait_idx(cb, i):
            pltpu.make_async_copy(src_hbm.at[pl.ds(base_w + cb, C)], S[i],
                                  I[i]).wait()
            pltpu.make_async_copy(dst_hbm.at[pl.ds(base_w + cb, C)], Dd[i],
                                  I[i]).wait()

        def start_gather(i, r):
            pltpu.async_copy(h_hbm.at[S[i]], R[r], G[r])

        def wait_gather(i, r):
            pltpu.make_async_copy(h_hbm.at[S[i]], R[r], G[r]).wait()

        def start_scatter(i, r):
            pltpu.async_copy(R[r], agg_sh.at[Dd[i]], T[r], add=True)

        def wait_scatter(i, r):
            pltpu.make_async_copy(R[r], agg_sh.at[Dd[i]], T[r]).wait()

        for i in range(K):
            fire_idx(i * C, i)

        @pl.loop(0, NROUND)
        def _round(rnd):
            cbase = rnd * (KI * C)
            for u in range(KI):
                r = u % K
                ipk = (u + K) % KI
                im1 = (u - 1) % KI
                rm1 = (u - 1) % K

                if u >= K:
                    wait_scatter(ipk, r)
                else:
                    @pl.when(rnd > 0)
                    def _(ipk=ipk, r=r):
                        wait_scatter(ipk, r)

                fire_idx(cbase + (u + K) * C, ipk)
                wait_idx(cbase + u * C, u)
                start_gather(u, r)

                if u == 0:
                    @pl.when(rnd > 0)
                    def _(im1=im1, rm1=rm1):
                        wait_gather(im1, rm1)
                        start_scatter(im1, rm1)
                else:
                    wait_gather(im1, rm1)
                    start_scatter(im1, rm1)

        EB = NROUND * KI * C
        for e in range(NCHUNK - NROUND * KI):
            r = e % K
            i = e % KI
            ipk = (e + K) % KI
            wait_scatter(ipk, r)
            if e == 0:
                fire_idx(EB + (e + K) * C, ipk)
            wait_idx(EB + e * C, i)
            start_gather(i, r)
            im1 = (e - 1) % KI if e > 0 else (NROUND * KI - 1) % KI
            rm1 = (e - 1) % K
            wait_gather(im1, rm1)
            start_scatter(im1, rm1)

        wait_gather(4, 0)
        start_scatter(4, 0)
        wait_scatter(1, 1)
        wait_scatter(2, 2)
        wait_scatter(3, 3)
        wait_scatter(4, 0)

        plsc.subcore_barrier()

        @pl.when(sid == 0)
        def _():
            pltpu.sync_copy(agg_sh, out_hbm.at[cid])

    return k(h, src, dst, zrows)


BN = 1000


def _leaky(v):
    return jnp.where(v > 0, v, 0.2 * v)


def _mlp_body(h_ref, agg_ref, wa_ref, ba_ref, wb_ref, bb_ref, o_ref):
    z = h_ref[...] + agg_ref[0] + agg_ref[1]
    z = _leaky(jnp.dot(z, wa_ref[...], preferred_element_type=jnp.float32,
                       precision=lax.Precision.HIGHEST) + ba_ref[...])
    z = _leaky(jnp.dot(z, wb_ref[...], preferred_element_type=jnp.float32,
                       precision=lax.Precision.HIGHEST) + bb_ref[...])
    o_ref[...] = z


def _tc_mlp(h, agg, waT, ba, wbT, bb):
    return pl.pallas_call(
        _mlp_body,
        grid=(N // BN,),
        in_specs=[
            pl.BlockSpec((BN, D), lambda i: (i, 0)),
            pl.BlockSpec((NC, BN, D), lambda i: (0, i, 0)),
            pl.BlockSpec((D, D), lambda i: (0, 0)),
            pl.BlockSpec((1, D), lambda i: (0, 0)),
            pl.BlockSpec((D, D), lambda i: (0, 0)),
            pl.BlockSpec((1, D), lambda i: (0, 0)),
        ],
        out_specs=pl.BlockSpec((BN, D), lambda i: (i, 0)),
        out_shape=jax.ShapeDtypeStruct((N, D), jnp.float32),
    )(h, agg, waT, ba, wbT, bb)


def _final_body(h_ref, agg_ref, wa_ref, ba_ref, wb_ref, bb_ref, w3_ref, b3_ref,
                o_ref):
    z = h_ref[...] + agg_ref[0] + agg_ref[1]
    z = _leaky(jnp.dot(z, wa_ref[...], preferred_element_type=jnp.float32,
                       precision=lax.Precision.HIGHEST) + ba_ref[...])
    z = _leaky(jnp.dot(z, wb_ref[...], preferred_element_type=jnp.float32,
                       precision=lax.Precision.HIGHEST) + bb_ref[...])
    o_ref[...] = jnp.sum(z * w3_ref[...], axis=1, keepdims=True) + b3_ref[...]


def _tc_final(h, agg, waT, ba, wbT, bb, w3, b3):
    return pl.pallas_call(
        _final_body,
        grid=(N // BN,),
        in_specs=[
            pl.BlockSpec((BN, D), lambda i: (i, 0)),
            pl.BlockSpec((NC, BN, D), lambda i: (0, i, 0)),
            pl.BlockSpec((D, D), lambda i: (0, 0)),
            pl.BlockSpec((1, D), lambda i: (0, 0)),
            pl.BlockSpec((D, D), lambda i: (0, 0)),
            pl.BlockSpec((1, D), lambda i: (0, 0)),
            pl.BlockSpec((1, D), lambda i: (0, 0)),
            pl.BlockSpec((1, 1), lambda i: (0, 0)),
        ],
        out_specs=pl.BlockSpec((BN, 1), lambda i: (i, 0)),
        out_shape=jax.ShapeDtypeStruct((N, 1), jnp.float32),
    )(h, agg, waT, ba, wbT, bb, w3, b3)


def kernel(x, adj, edge_index, W1a, b1a, W1b, b1b, W2a, b2a, W2b, b2b, W3, b3):
    src = edge_index[0]
    dst = edge_index[1]
    z0 = jnp.zeros((NP, D), jnp.float32)
    agg1 = _sc_agg(x, src, dst, z0)
    h1 = _tc_mlp(x, agg1, W1a.T, b1a.reshape(1, D), W1b.T, b1b.reshape(1, D))
    agg2 = _sc_agg(h1, src, dst, z0)
    return _tc_final(h1, agg2, W2a.T, b2a.reshape(1, D), W2b.T, b2b.reshape(1, D),
                     W3, b3.reshape(1, 1))


# speedup vs baseline: 12.3459x; 1.1917x over previous
import functools

import jax
import jax.numpy as jnp
from jax import lax
from jax.experimental import pallas as pl
from jax.experimental.pallas import tpu as pltpu
from jax.experimental.pallas import tpu_sc as plsc

N, E, D = 10000, 320000, 128
NC, NS = 2, 16
NW = NC * NS
EPW = E // NW
C = 80
NCHUNK = EPW // C
NP = 10240
K = 4
KI = 2 * K
NROUND = NCHUNK // KI

_mesh = plsc.VectorSubcoreMesh(core_axis_name="c", subcore_axis_name="s")


def _sc_agg(h, ei_flat, zrows):
    @functools.partial(
        pl.kernel,
        out_type=jax.ShapeDtypeStruct((NC, NP, D), jnp.float32),
        mesh=_mesh,
        scratch_types=(
            [pltpu.VMEM((C,), jnp.int32)] * KI
            + [pltpu.VMEM((C,), jnp.int32)] * KI
            + [pltpu.VMEM((C, D), jnp.float32)] * K
            + [pltpu.VMEM_SHARED((NP, D), jnp.float32)]
            + [pltpu.SemaphoreType.DMA] * (1 + KI + 2 * K)
        ),
    )
    def k(h_hbm, ei_hbm, z_hbm, out_hbm, *refs):
        S = refs[0:KI]
        Dd = refs[KI:2 * KI]
        R = refs[2 * KI:2 * KI + K]
        agg_sh = refs[2 * KI + K]
        gz = refs[2 * KI + K + 1]
        base_sem = 2 * KI + K + 2
        I = refs[base_sem:base_sem + KI]
        G = refs[base_sem + KI:base_sem + KI + K]
        T = refs[base_sem + KI + K:base_sem + KI + 2 * K]

        cid = lax.axis_index("c")
        sid = lax.axis_index("s")
        wid = sid * NC + cid
        base_w = wid * EPW

        @pl.when(sid == 0)
        def _():
            pltpu.async_copy(z_hbm, agg_sh, gz)
            pltpu.make_async_copy(z_hbm, agg_sh, gz).wait()

        plsc.subcore_barrier()

        def fire_idx(cb, i):
            pltpu.async_copy(ei_hbm.at[pl.ds(base_w + cb, C)], S[i], I[i])
            pltpu.async_copy(ei_hbm.at[pl.ds(E + base_w + cb, C)], Dd[i], I[i])

        def wait_idx(cb, i):
            pltpu.make_async_copy(ei_hbm.at[pl.ds(base_w + cb, C)], S[i],
                                  I[i]).wait()
            pltpu.make_async_copy(ei_hbm.at[pl.ds(E + base_w + cb, C)], Dd[i],
                                  I[i]).wait()

        def start_gather(i, r):
            pltpu.async_copy(h_hbm.at[S[i]], R[r], G[r])

        def wait_gather(i, r):
            pltpu.make_async_copy(h_hbm.at[S[i]], R[r], G[r]).wait()

        def start_scatter(i, r):
            pltpu.async_copy(R[r], agg_sh.at[Dd[i]], T[r], add=True)

        def wait_scatter(i, r):
            pltpu.make_async_copy(R[r], agg_sh.at[Dd[i]], T[r]).wait()

        for i in range(K):
            fire_idx(i * C, i)

        @pl.loop(0, NROUND)
        def _round(rnd):
            cbase = rnd * (KI * C)
            for u in range(KI):
                r = u % K
                ipk = (u + K) % KI
                im1 = (u - 1) % KI
                rm1 = (u - 1) % K

                if u >= K:
                    wait_scatter(ipk, r)
                else:
                    @pl.when(rnd > 0)
                    def _(ipk=ipk, r=r):
                        wait_scatter(ipk, r)

                fire_idx(cbase + (u + K) * C, ipk)
                wait_idx(cbase + u * C, u)
                start_gather(u, r)

                if u == 0:
                    @pl.when(rnd > 0)
                    def _(im1=im1, rm1=rm1):
                        wait_gather(im1, rm1)
                        start_scatter(im1, rm1)
                else:
                    wait_gather(im1, rm1)
                    start_scatter(im1, rm1)

        EB = NROUND * KI * C
        for e in range(NCHUNK - NROUND * KI):
            r = e % K
            i = e % KI
            ipk = (e + K) % KI
            wait_scatter(ipk, r)
            if e == 0:
                fire_idx(EB + (e + K) * C, ipk)
            wait_idx(EB + e * C, i)
            start_gather(i, r)
            im1 = (e - 1) % KI if e > 0 else (NROUND * KI - 1) % KI
            rm1 = (e - 1) % K
            wait_gather(im1, rm1)
            start_scatter(im1, rm1)

        wait_gather(4, 0)
        start_scatter(4, 0)
        wait_scatter(1, 1)
        wait_scatter(2, 2)
        wait_scatter(3, 3)
        wait_scatter(4, 0)

        plsc.subcore_barrier()

        @pl.when(sid == 0)
        def _():
            pltpu.sync_copy(agg_sh, out_hbm.at[cid])

    return k(h, ei_flat, zrows)


BN = 1000


def _leaky(v):
    return jnp.where(v > 0, v, 0.2 * v)


def _dot_wT(z, w):
    return lax.dot_general(z, w, (((1,), (1,)), ((), ())),
                           preferred_element_type=jnp.float32,
                           precision=lax.Precision.DEFAULT)


def _mlp_body(h_ref, agg_ref, wa_ref, ba_ref, wb_ref, bb_ref, o_ref):
    z = h_ref[...] + agg_ref[0] + agg_ref[1]
    z = _leaky(_dot_wT(z, wa_ref[...]) + ba_ref[...])
    z = _leaky(_dot_wT(z, wb_ref[...]) + bb_ref[...])
    o_ref[...] = z


def _tc_mlp(h, agg, waT, ba, wbT, bb):
    return pl.pallas_call(
        _mlp_body,
        grid=(N // BN,),
        in_specs=[
            pl.BlockSpec((BN, D), lambda i: (i, 0)),
            pl.BlockSpec((NC, BN, D), lambda i: (0, i, 0)),
            pl.BlockSpec((D, D), lambda i: (0, 0)),
            pl.BlockSpec((1, D), lambda i: (0, 0)),
            pl.BlockSpec((D, D), lambda i: (0, 0)),
            pl.BlockSpec((1, D), lambda i: (0, 0)),
        ],
        out_specs=pl.BlockSpec((BN, D), lambda i: (i, 0)),
        out_shape=jax.ShapeDtypeStruct((N, D), jnp.float32),
    )(h, agg, waT, ba, wbT, bb)


def _final_body(h_ref, agg_ref, wa_ref, ba_ref, wb_ref, bb_ref, w3_ref, b3_ref,
                o_ref):
    z = h_ref[...] + agg_ref[0] + agg_ref[1]
    z = _leaky(_dot_wT(z, wa_ref[...]) + ba_ref[...])
    z = _leaky(_dot_wT(z, wb_ref[...]) + bb_ref[...])
    o_ref[...] = jnp.sum(z * w3_ref[...], axis=1, keepdims=True) + b3_ref[...]


def _tc_final(h, agg, waT, ba, wbT, bb, w3, b3):
    return pl.pallas_call(
        _final_body,
        grid=(N // BN,),
        in_specs=[
            pl.BlockSpec((BN, D), lambda i: (i, 0)),
            pl.BlockSpec((NC, BN, D), lambda i: (0, i, 0)),
            pl.BlockSpec((D, D), lambda i: (0, 0)),
            pl.BlockSpec((1, D), lambda i: (0, 0)),
            pl.BlockSpec((D, D), lambda i: (0, 0)),
            pl.BlockSpec((1, D), lambda i: (0, 0)),
            pl.BlockSpec((1, D), lambda i: (0, 0)),
            pl.BlockSpec((1, 1), lambda i: (0, 0)),
        ],
        out_specs=pl.BlockSpec((BN, 1), lambda i: (i, 0)),
        out_shape=jax.ShapeDtypeStruct((N, 1), jnp.float32),
    )(h, agg, waT, ba, wbT, bb, w3, b3)


def kernel(x, adj, edge_index, W1a, b1a, W1b, b1b, W2a, b2a, W2b, b2b, W3, b3):
    z0 = jnp.zeros((NP, D), jnp.float32)
    ei = edge_index.reshape(2 * E)
    agg1 = _sc_agg(x, ei, z0)
    h1 = _tc_mlp(x, agg1, W1a, b1a.reshape(1, D), W1b, b1b.reshape(1, D))
    agg2 = _sc_agg(h1, ei, z0)
    return _tc_final(h1, agg2, W2a, b2a.reshape(1, D), W2b, b2b.reshape(1, D),
                     W3, b3.reshape(1, 1))


# speedup vs baseline: 12.6160x; 1.0219x over previous
import functools

import jax
import jax.numpy as jnp
from jax import lax
from jax.experimental import pallas as pl
from jax.experimental.pallas import tpu as pltpu
from jax.experimental.pallas import tpu_sc as plsc

N, E, D = 10000, 320000, 128
NC, NS = 2, 16
NW = NC * NS
EPW = E // NW
C = 80
NCHUNK = EPW // C
NP = 10240
K = 4
KI = 2 * K
NROUND = NCHUNK // KI

_mesh = plsc.VectorSubcoreMesh(core_axis_name="c", subcore_axis_name="s")


def _sc_agg(h, ei_flat, zrows):
    @functools.partial(
        pl.kernel,
        out_type=jax.ShapeDtypeStruct((NC, NP, D), jnp.float32),
        mesh=_mesh,
        scratch_types=(
            [pltpu.VMEM((C,), jnp.int32)] * KI
            + [pltpu.VMEM((C,), jnp.int32)] * KI
            + [pltpu.VMEM((C, D), jnp.float32)] * K
            + [pltpu.VMEM_SHARED((NP, D), jnp.float32)]
            + [pltpu.SemaphoreType.DMA] * (1 + KI + 2 * K)
        ),
    )
    def k(h_hbm, ei_hbm, z_hbm, out_hbm, *refs):
        S = refs[0:KI]
        Dd = refs[KI:2 * KI]
        R = refs[2 * KI:2 * KI + K]
        agg_sh = refs[2 * KI + K]
        gz = refs[2 * KI + K + 1]
        base_sem = 2 * KI + K + 2
        I = refs[base_sem:base_sem + KI]
        G = refs[base_sem + KI:base_sem + KI + K]
        T = refs[base_sem + KI + K:base_sem + KI + 2 * K]

        cid = lax.axis_index("c")
        sid = lax.axis_index("s")
        wid = sid * NC + cid
        base_w = wid * EPW

        @pl.when(sid == 0)
        def _():
            pltpu.async_copy(z_hbm, agg_sh, gz)
            pltpu.make_async_copy(z_hbm, agg_sh, gz).wait()

        plsc.subcore_barrier()

        def fire_idx(cb, i):
            pltpu.async_copy(ei_hbm.at[pl.ds(base_w + cb, C)], S[i], I[i])
            pltpu.async_copy(ei_hbm.at[pl.ds(E + base_w + cb, C)], Dd[i], I[i])

        def wait_idx(cb, i):
            pltpu.make_async_copy(ei_hbm.at[pl.ds(base_w + cb, C)], S[i],
                                  I[i]).wait()
            pltpu.make_async_copy(ei_hbm.at[pl.ds(E + base_w + cb, C)], Dd[i],
                                  I[i]).wait()

        def start_gather(i, r):
            pltpu.async_copy(h_hbm.at[S[i]], R[r], G[r])

        def wait_gather(i, r):
            pltpu.make_async_copy(h_hbm.at[S[i]], R[r], G[r]).wait()

        def start_scatter(i, r):
            pltpu.async_copy(R[r], agg_sh.at[Dd[i]], T[r], add=True)

        def wait_scatter(i, r):
            pltpu.make_async_copy(R[r], agg_sh.at[Dd[i]], T[r]).wait()

        for i in range(K):
            fire_idx(i * C, i)

        @pl.loop(0, NROUND)
        def _round(rnd):
            cbase = rnd * (KI * C)
            for u in range(KI):
                r = u % K
                ipk = (u + K) % KI
                im1 = (u - 1) % KI
                rm1 = (u - 1) % K

                if u >= K:
                    wait_scatter(ipk, r)
                else:
                    @pl.when(rnd > 0)
                    def _(ipk=ipk, r=r):
                        wait_scatter(ipk, r)

                fire_idx(cbase + (u + K) * C, ipk)
                wait_idx(cbase + u * C, u)
                start_gather(u, r)

                if u == 0:
                    @pl.when(rnd > 0)
                    def _(im1=im1, rm1=rm1):
                        wait_gather(im1, rm1)
                        start_scatter(im1, rm1)
                else:
                    wait_gather(im1, rm1)
                    start_scatter(im1, rm1)

        EB = NROUND * KI * C
        for e in range(NCHUNK - NROUND * KI):
            r = e % K
            i = e % KI
            ipk = (e + K) % KI
            wait_scatter(ipk, r)
            if e == 0:
                fire_idx(EB + (e + K) * C, ipk)
            wait_idx(EB + e * C, i)
            start_gather(i, r)
            im1 = (e - 1) % KI if e > 0 else (NROUND * KI - 1) % KI
            rm1 = (e - 1) % K
            wait_gather(im1, rm1)
            start_scatter(im1, rm1)

        wait_gather(4, 0)
        start_scatter(4, 0)
        wait_scatter(1, 1)
        wait_scatter(2, 2)
        wait_scatter(3, 3)
        wait_scatter(4, 0)

        plsc.subcore_barrier()

        @pl.when(sid == 0)
        def _():
            pltpu.sync_copy(agg_sh, out_hbm.at[cid])

    return k(h, ei_flat, zrows)


BN = 2000


def _leaky(v):
    return jnp.where(v > 0, v, 0.2 * v)


def _dot_wT(z, w):
    return lax.dot_general(z, w, (((1,), (1,)), ((), ())),
                           preferred_element_type=jnp.float32,
                           precision=lax.Precision.DEFAULT)


def _mlp_body(h_ref, agg_ref, wa_ref, ba_ref, wb_ref, bb_ref, o_ref):
    z = h_ref[...] + agg_ref[0] + agg_ref[1]
    z = _leaky(_dot_wT(z, wa_ref[...]) + ba_ref[...])
    z = _leaky(_dot_wT(z, wb_ref[...]) + bb_ref[...])
    o_ref[...] = z


def _tc_mlp(h, agg, waT, ba, wbT, bb):
    return pl.pallas_call(
        _mlp_body,
        grid=(N // BN,),
        in_specs=[
            pl.BlockSpec((BN, D), lambda i: (i, 0)),
            pl.BlockSpec((NC, BN, D), lambda i: (0, i, 0)),
            pl.BlockSpec((D, D), lambda i: (0, 0)),
            pl.BlockSpec((1, D), lambda i: (0, 0)),
            pl.BlockSpec((D, D), lambda i: (0, 0)),
            pl.BlockSpec((1, D), lambda i: (0, 0)),
        ],
        out_specs=pl.BlockSpec((BN, D), lambda i: (i, 0)),
        out_shape=jax.ShapeDtypeStruct((N, D), jnp.float32),
    )(h, agg, waT, ba, wbT, bb)


def _final_body(h_ref, agg_ref, wa_ref, ba_ref, wb_ref, bb_ref, w3_ref, b3_ref,
                o_ref):
    z = h_ref[...] + agg_ref[0] + agg_ref[1]
    z = _leaky(_dot_wT(z, wa_ref[...]) + ba_ref[...])
    z = _leaky(_dot_wT(z, wb_ref[...]) + bb_ref[...])
    o_ref[...] = jnp.sum(z * w3_ref[...], axis=1, keepdims=True) + b3_ref[...]


def _tc_final(h, agg, waT, ba, wbT, bb, w3, b3):
    return pl.pallas_call(
        _final_body,
        grid=(N // BN,),
        in_specs=[
            pl.BlockSpec((BN, D), lambda i: (i, 0)),
            pl.BlockSpec((NC, BN, D), lambda i: (0, i, 0)),
            pl.BlockSpec((D, D), lambda i: (0, 0)),
            pl.BlockSpec((1, D), lambda i: (0, 0)),
            pl.BlockSpec((D, D), lambda i: (0, 0)),
            pl.BlockSpec((1, D), lambda i: (0, 0)),
            pl.BlockSpec((1, D), lambda i: (0, 0)),
            pl.BlockSpec((1, 1), lambda i: (0, 0)),
        ],
        out_specs=pl.BlockSpec((BN, 1), lambda i: (i, 0)),
        out_shape=jax.ShapeDtypeStruct((N, 1), jnp.float32),
    )(h, agg, waT, ba, wbT, bb, w3, b3)


def kernel(x, adj, edge_index, W1a, b1a, W1b, b1b, W2a, b2a, W2b, b2b, W3, b3):
    z0 = jnp.zeros((NP, D), jnp.float32)
    ei = edge_index.reshape(2 * E)
    agg1 = _sc_agg(x, ei, z0)
    h1 = _tc_mlp(x, agg1, W1a, b1a.reshape(1, D), W1b, b1b.reshape(1, D))
    agg2 = _sc_agg(h1, ei, z0)
    return _tc_final(h1, agg2, W2a, b2a.reshape(1, D), W2b, b2b.reshape(1, D),
                     W3, b3.reshape(1, 1))
